# Initial kernel scaffold; baseline (speedup 1.0000x reference)
import jax
import jax.numpy as jnp
from jax.experimental import pallas as pl


def kernel(query, reference_points, input_flatten, input_spatial_shapes, input_level_start_index, params):
    raise NotImplementedError("write your pallas kernel here")


# speedup vs baseline: 7.3867x; 7.3867x over previous
import functools

import jax
import jax.numpy as jnp
import numpy as np
from jax import lax
from jax.experimental import pallas as pl
from jax.experimental.pallas import tpu as pltpu
from jax.experimental.pallas import tpu_sc as plsc

D_MODEL = 256
N_HEADS = 8
N_LEVELS = 4
N_POINTS = 4
LEVEL_WH = (128, 64, 32, 16)
LEVEL_START = (0, 16384, 20480, 21504)
LEN_IN = 21760
B = 4
LQ = 300
BQ = B * LQ
SUFS = ("g", "head", "lt", "rt", "ul", "ll")
BOXC = {
    "g": (0.5, 0.5, 1.0, 1.0),
    "head": (0.5, 0.115, 0.7, 0.23),
    "lt": (0.25, 0.41, 0.5, 0.36),
    "rt": (0.75, 0.41, 0.5, 0.36),
    "ul": (0.5, 0.655, 0.7, 0.23),
    "ll": (0.5, 0.885, 0.7, 0.23),
}
NB = len(SUFS)
N_ITEMS = NB * BQ
N_WORKERS = 32
ITEMS_PER_WORKER = N_ITEMS // N_WORKERS
N_TABLE_ROWS = B * LEN_IN * NB * N_HEADS


def _build_consts():
    perm_off = np.zeros(256, np.int64)
    for col in range(256):
        xy = col // 128
        r = col % 128
        l, hp = r // 32, r % 32
        h, p = hp // 4, hp % 4
        perm_off[col] = h * 32 + l * 8 + p * 2 + xy
    perm_aw = np.zeros(128, np.int64)
    for j in range(128):
        l, hp = j // 32, j % 32
        h, p = hp // 4, hp % 4
        perm_aw[j] = h * 16 + l * 4 + p
    s_mat = np.zeros((128, 128), np.float32)
    for j in range(128):
        for jp in range(128):
            if (j % 32) // 4 == (jp % 32) // 4:
                s_mat[jp, j] = 1.0
    cref = np.zeros((NB, 16, 512), np.float32)
    for bi, suf in enumerate(SUFS):
        ax, ay, sw, sh = BOXC[suf]
        m = np.zeros((4, 4), np.float32)
        m[0, 0] = 1.0
        m[1, 1] = 1.0
        m[2, 0] = ax - 0.5
        m[3, 1] = ay - 0.5
        m[2, 2] = sw
        m[3, 3] = sh
        for k in range(4):
            for j in range(128):
                l = j // 32
                for mm in range(4):
                    cref[bi, l * 4 + mm, k * 128 + j] = m[mm, k]
    return perm_off, perm_aw, s_mat, cref


PERM_OFF, PERM_AW, S_MAT, CREF = _build_consts()


def _prep_body(q_ref, ref_ref, wq_ref, bq_ref, woff_ref, boff_ref, waw_ref,
               baw_ref, s_ref, cref_ref, idx_ref, w_ref):
    bi = pl.program_id(0)
    f32 = jnp.float32
    q = q_ref[...]
    qb = jnp.dot(q, wq_ref[0], preferred_element_type=f32, precision=lax.Precision.HIGHEST) + bq_ref[0]
    offp = jnp.dot(qb, woff_ref[0], preferred_element_type=f32, precision=lax.Precision.HIGHEST) + boff_ref[0]
    awl = jnp.dot(qb, waw_ref[0], preferred_element_type=f32, precision=lax.Precision.HIGHEST) + baw_ref[0]
    mx = jnp.max(awl, axis=1, keepdims=True)
    e = jnp.exp(awl - mx)
    den = jnp.dot(e, s_ref[...], preferred_element_type=f32, precision=lax.Precision.HIGHEST)
    awn = e / den
    refq = jnp.dot(ref_ref[...], cref_ref[0], preferred_element_type=f32, precision=lax.Precision.HIGHEST)
    cx, cy = refq[:, 0:128], refq[:, 128:256]
    rw, rh = refq[:, 256:384], refq[:, 384:512]
    offx, offy = offp[:, 0:128], offp[:, 128:256]

    li = lax.broadcasted_iota(jnp.int32, (BQ, 128), 1)
    lvl = lax.shift_right_logical(li, 5)
    wv = lax.shift_right_logical(jnp.full_like(li, 128), lvl)
    startv = jnp.where(lvl == 0, LEVEL_START[0],
                       jnp.where(lvl == 1, LEVEL_START[1],
                                 jnp.where(lvl == 2, LEVEL_START[2],
                                           LEVEL_START[3])))
    headv = lax.shift_right_logical(li & 31, 2)
    bv = lax.broadcasted_iota(jnp.int32, (BQ, 128), 0) // LQ
    wf = wv.astype(f32)

    locx = cx + offx / N_POINTS * rw * 0.5
    locy = cy + offy / N_POINTS * rh * 0.5
    x = locx * wf - 0.5
    y = locy * wf - 0.5
    x0 = jnp.floor(x)
    y0 = jnp.floor(y)
    lw = x - x0
    lh = y - y0
    x0i = x0.astype(jnp.int32)
    y0i = y0.astype(jnp.int32)
    base_row = bv * LEN_IN

    for c, (dy, dx) in enumerate(((0, 0), (0, 1), (1, 0), (1, 1))):
        yy = y0i + dy
        xx = x0i + dx
        wyf = lh if dy else 1.0 - lh
        wxf = lw if dx else 1.0 - lw
        valid = ((yy >= 0) & (yy < wv) & (xx >= 0) & (xx < wv)).astype(f32)
        yc = jnp.clip(yy, 0, wv - 1)
        xc = jnp.clip(xx, 0, wv - 1)
        flat = yc * wv + xc + startv
        gidx = (base_row + flat) * (NB * N_HEADS) + bi * N_HEADS + headv
        idx_ref[0, :, c * 128:(c + 1) * 128] = gidx
        w_ref[0, :, c * 128:(c + 1) * 128] = wyf * wxf * valid * awn


def _run_prep(q2, ref16, wq_s, bq_s, woff_s, boff_s, waw_s, baw_s, s_mat, cref):
    return pl.pallas_call(
        _prep_body,
        grid=(NB,),
        in_specs=[
            pl.BlockSpec((BQ, 256), lambda i: (0, 0)),
            pl.BlockSpec((BQ, 16), lambda i: (0, 0)),
            pl.BlockSpec((1, 256, 256), lambda i: (i, 0, 0)),
            pl.BlockSpec((1, 1, 256), lambda i: (i, 0, 0)),
            pl.BlockSpec((1, 256, 256), lambda i: (i, 0, 0)),
            pl.BlockSpec((1, 1, 256), lambda i: (i, 0, 0)),
            pl.BlockSpec((1, 256, 128), lambda i: (i, 0, 0)),
            pl.BlockSpec((1, 1, 128), lambda i: (i, 0, 0)),
            pl.BlockSpec((128, 128), lambda i: (0, 0)),
            pl.BlockSpec((1, 16, 512), lambda i: (i, 0, 0)),
        ],
        out_specs=[
            pl.BlockSpec((1, BQ, 512), lambda i: (i, 0, 0)),
            pl.BlockSpec((1, BQ, 512), lambda i: (i, 0, 0)),
        ],
        out_shape=[
            jax.ShapeDtypeStruct((NB, BQ, 512), jnp.int32),
            jax.ShapeDtypeStruct((NB, BQ, 512), jnp.float32),
        ],
    )(q2, ref16, wq_s, bq_s, woff_s, boff_s, waw_s, baw_s, s_mat, cref)


_VM_ROWS = 512
_VM_GRID = (B * LEN_IN) // _VM_ROWS


def _vmat_body(x_ref, w_ref, b_ref, o_ref):
    o_ref[...] = (jnp.dot(x_ref[...], w_ref[...],
                          preferred_element_type=jnp.float32) + b_ref[...])


def _run_vmat(x, wcat, bcat):
    return pl.pallas_call(
        _vmat_body,
        grid=(_VM_GRID,),
        in_specs=[
            pl.BlockSpec((_VM_ROWS, 256), lambda i: (i, 0)),
            pl.BlockSpec((256, 1536), lambda i: (0, 0)),
            pl.BlockSpec((1, 1536), lambda i: (0, 0)),
        ],
        out_specs=pl.BlockSpec((_VM_ROWS, 1536), lambda i: (i, 0)),
        out_shape=jax.ShapeDtypeStruct((B * LEN_IN, 1536), jnp.float32),
    )(x, wcat, bcat)


def _gather_body(table_ref, idx_ref, w_ref, out_ref, idx_v, w_v, rows_v,
                 out_v, sem):
    wid = lax.axis_index("s") * 2 + lax.axis_index("c")
    base = wid * ITEMS_PER_WORKER

    def item(it, carry):
        g = base + it
        pltpu.sync_copy(idx_ref.at[g], idx_v)
        pltpu.sync_copy(w_ref.at[g], w_v)
        descs = [
            pltpu.async_copy(table_ref.at[idx_v.at[c]], rows_v.at[c], sem)
            for c in range(4)
        ]
        for d in descs:
            d.wait()
        acc = [jnp.zeros((16,), jnp.float32) for _ in range(16)]
        for c in range(4):
            for gq in range(8):
                wv = w_v[pl.ds(c * 128 + gq * 16, 16)]
                for t in range(16):
                    jb = gq * 16 + t
                    h = (jb >> 2) & 7
                    wb = jnp.full((16,), wv[t])
                    r0 = rows_v[c, jb, pl.ds(0, 16)]
                    r1 = rows_v[c, jb, pl.ds(16, 16)]
                    acc[2 * h] = acc[2 * h] + r0 * wb
                    acc[2 * h + 1] = acc[2 * h + 1] + r1 * wb
        for h in range(8):
            out_v[h, pl.ds(0, 16)] = acc[2 * h]
            out_v[h, pl.ds(16, 16)] = acc[2 * h + 1]
        pltpu.sync_copy(out_v, out_ref.at[g])
        return carry

    lax.fori_loop(0, ITEMS_PER_WORKER, item, 0)


def _run_gather(table, idx3, w3):
    mesh = plsc.VectorSubcoreMesh(core_axis_name="c", subcore_axis_name="s")
    fn = pl.kernel(
        _gather_body,
        out_type=jax.ShapeDtypeStruct((N_ITEMS, 8, 32), jnp.float32),
        mesh=mesh,
        compiler_params=pltpu.CompilerParams(needs_layout_passes=False,
                                             use_tc_tiling_on_sc=False),
        scratch_types=[
            pltpu.VMEM((4, 128), jnp.int32),
            pltpu.VMEM((512,), jnp.float32),
            pltpu.VMEM((4, 128, 32), jnp.float32),
            pltpu.VMEM((8, 32), jnp.float32),
            pltpu.SemaphoreType.DMA,
        ],
    )
    return fn(table, idx3, w3)


def _final_body(q_ref, g_ref, wout_ref, bout_ref, wpw_ref, bpw_ref, woc_ref,
                boc_ref, o_ref):
    f32 = jnp.float32
    q = q_ref[...]
    pwl = jnp.dot(q, wpw_ref[...], preferred_element_type=f32) + bpw_ref[...]
    pm = jnp.max(pwl, axis=1, keepdims=True)
    pe = jnp.exp(pwl - pm)
    pw = pe / jnp.sum(pe, axis=1, keepdims=True)
    acc = None
    for bi in range(NB):
        ob = jnp.dot(g_ref[bi], wout_ref[bi],
                     preferred_element_type=f32) + bout_ref[bi]
        t = pw[:, bi:bi + 1] * ob
        acc = t if acc is None else acc + t
    o_ref[...] = jnp.dot(acc, woc_ref[...],
                         preferred_element_type=f32) + boc_ref[...]


def _run_final(q2, g6, wout_s, bout_s, wpw_t, bpw_p, woc_t, boc2):
    return pl.pallas_call(
        _final_body,
        out_shape=jax.ShapeDtypeStruct((BQ, 256), jnp.float32),
    )(q2, g6, wout_s, bout_s, wpw_t, bpw_p, woc_t, boc2)


def kernel(query, reference_points, input_flatten, input_spatial_shapes,
           input_level_start_index, params):
    del input_spatial_shapes, input_level_start_index
    f32 = jnp.float32
    q2 = query.reshape(BQ, 256)
    ref16 = reference_points.reshape(BQ, 16)
    x = input_flatten.reshape(B * LEN_IN, 256)

    eye = jnp.asarray(np.eye(256, dtype=np.float32))
    wq_s = jnp.stack([eye] + [params["W_q_" + s].T for s in SUFS[1:]])
    bq_s = jnp.stack([jnp.zeros((256,), f32)] +
                     [params["b_q_" + s] for s in SUFS[1:]]).reshape(NB, 1, 256)
    woff_s = jnp.stack([params["W_off_" + s][PERM_OFF].T for s in SUFS])
    boff_s = jnp.stack([params["b_off_" + s][PERM_OFF]
                        for s in SUFS]).reshape(NB, 1, 256)
    waw_s = jnp.stack([params["W_aw_" + s][PERM_AW].T for s in SUFS])
    baw_s = jnp.stack([params["b_aw_" + s][PERM_AW]
                       for s in SUFS]).reshape(NB, 1, 128)
    idx, w = _run_prep(q2, ref16, wq_s, bq_s, woff_s, boff_s, waw_s, baw_s,
                       jnp.asarray(S_MAT), jnp.asarray(CREF))

    wcat = jnp.concatenate([params["W_val_" + s] for s in SUFS], axis=0).T
    bcat = jnp.concatenate([params["b_val_" + s]
                            for s in SUFS]).reshape(1, 1536)
    v = _run_vmat(x, wcat, bcat)
    table = v.reshape(N_TABLE_ROWS, 32)

    idx3 = idx.reshape(N_ITEMS, 4, 128)
    w2 = w.reshape(N_ITEMS, 512)
    g = _run_gather(table, idx3, w2)
    g6 = g.reshape(NB, BQ, 256)

    wout_s = jnp.stack([params["W_out_" + s].T for s in SUFS])
    bout_s = jnp.stack([params["b_out_" + s] for s in SUFS]).reshape(NB, 1, 256)
    wpw_t = jnp.concatenate([params["W_pw"].T, jnp.zeros((256, 2), f32)],
                            axis=1)
    bpw_p = jnp.concatenate([params["b_pw"],
                             jnp.full((2,), -1e30, f32)]).reshape(1, 8)
    woc_t = params["W_oc"].T
    boc2 = params["b_oc"].reshape(1, 256)
    out = _run_final(q2, g6, wout_s, bout_s, wpw_t, bpw_p, woc_t, boc2)
    return out.reshape(B, LQ, 256)


# speedup vs baseline: 10.4776x; 1.4184x over previous
import functools

import jax
import jax.numpy as jnp
import numpy as np
from jax import lax
from jax.experimental import pallas as pl
from jax.experimental.pallas import tpu as pltpu
from jax.experimental.pallas import tpu_sc as plsc

D_MODEL = 256
N_HEADS = 8
N_LEVELS = 4
N_POINTS = 4
LEVEL_WH = (128, 64, 32, 16)
LEVEL_START = (0, 16384, 20480, 21504)
LEN_IN = 21760
B = 4
LQ = 300
BQ = B * LQ
SUFS = ("g", "head", "lt", "rt", "ul", "ll")
BOXC = {
    "g": (0.5, 0.5, 1.0, 1.0),
    "head": (0.5, 0.115, 0.7, 0.23),
    "lt": (0.25, 0.41, 0.5, 0.36),
    "rt": (0.75, 0.41, 0.5, 0.36),
    "ul": (0.5, 0.655, 0.7, 0.23),
    "ll": (0.5, 0.885, 0.7, 0.23),
}
NB = len(SUFS)
N_ITEMS = NB * BQ
N_WORKERS = 32
ITEMS_PER_WORKER = N_ITEMS // N_WORKERS
N_TABLE_ROWS = B * LEN_IN * NB * N_HEADS


def _build_consts():
    perm_off = np.zeros(256, np.int64)
    for col in range(256):
        xy = col // 128
        r = col % 128
        l, hp = r // 32, r % 32
        h, p = hp // 4, hp % 4
        perm_off[col] = h * 32 + l * 8 + p * 2 + xy
    perm_aw = np.zeros(128, np.int64)
    for j in range(128):
        l, hp = j // 32, j % 32
        h, p = hp // 4, hp % 4
        perm_aw[j] = h * 16 + l * 4 + p
    s_mat = np.zeros((128, 128), np.float32)
    for j in range(128):
        for jp in range(128):
            if (j % 32) // 4 == (jp % 32) // 4:
                s_mat[jp, j] = 1.0
    cref = np.zeros((NB, 16, 512), np.float32)
    for bi, suf in enumerate(SUFS):
        ax, ay, sw, sh = BOXC[suf]
        m = np.zeros((4, 4), np.float32)
        m[0, 0] = 1.0
        m[1, 1] = 1.0
        m[2, 0] = ax - 0.5
        m[3, 1] = ay - 0.5
        m[2, 2] = sw
        m[3, 3] = sh
        for k in range(4):
            for j in range(128):
                l = j // 32
                for mm in range(4):
                    cref[bi, l * 4 + mm, k * 128 + j] = m[mm, k]
    return perm_off, perm_aw, s_mat, cref


PERM_OFF, PERM_AW, S_MAT, CREF = _build_consts()

OUT_PERM = np.zeros(256, np.int64)
for _h in range(8):
    for _k in range(16):
        OUT_PERM[_h * 32 + _k] = _h * 32 + 2 * _k
        OUT_PERM[_h * 32 + 16 + _k] = _h * 32 + 2 * _k + 1


def _prep_body(q_ref, ref_ref, wq_ref, bq_ref, woff_ref, boff_ref, waw_ref,
               baw_ref, s_ref, cref_ref, idx_ref, w_ref):
    bi = pl.program_id(0)
    f32 = jnp.float32
    q = q_ref[...]
    qb = jnp.dot(q, wq_ref[0], preferred_element_type=f32, precision=lax.Precision.HIGHEST) + bq_ref[0]
    offp = jnp.dot(qb, woff_ref[0], preferred_element_type=f32, precision=lax.Precision.HIGHEST) + boff_ref[0]
    awl = jnp.dot(qb, waw_ref[0], preferred_element_type=f32, precision=lax.Precision.HIGHEST) + baw_ref[0]
    mx = jnp.max(awl, axis=1, keepdims=True)
    e = jnp.exp(awl - mx)
    den = jnp.dot(e, s_ref[...], preferred_element_type=f32, precision=lax.Precision.HIGHEST)
    awn = e / den
    refq = jnp.dot(ref_ref[...], cref_ref[0], preferred_element_type=f32, precision=lax.Precision.HIGHEST)
    cx, cy = refq[:, 0:128], refq[:, 128:256]
    rw, rh = refq[:, 256:384], refq[:, 384:512]
    offx, offy = offp[:, 0:128], offp[:, 128:256]

    li = lax.broadcasted_iota(jnp.int32, (BQ, 128), 1)
    lvl = lax.shift_right_logical(li, 5)
    wv = lax.shift_right_logical(jnp.full_like(li, 128), lvl)
    startv = jnp.where(lvl == 0, LEVEL_START[0],
                       jnp.where(lvl == 1, LEVEL_START[1],
                                 jnp.where(lvl == 2, LEVEL_START[2],
                                           LEVEL_START[3])))
    headv = lax.shift_right_logical(li & 31, 2)
    bv = lax.broadcasted_iota(jnp.int32, (BQ, 128), 0) // LQ
    wf = wv.astype(f32)

    locx = cx + offx / N_POINTS * rw * 0.5
    locy = cy + offy / N_POINTS * rh * 0.5
    x = locx * wf - 0.5
    y = locy * wf - 0.5
    x0 = jnp.floor(x)
    y0 = jnp.floor(y)
    lw = x - x0
    lh = y - y0
    x0i = x0.astype(jnp.int32)
    y0i = y0.astype(jnp.int32)
    base_row = bv * LEN_IN

    for c, (dy, dx) in enumerate(((0, 0), (0, 1), (1, 0), (1, 1))):
        yy = y0i + dy
        xx = x0i + dx
        wyf = lh if dy else 1.0 - lh
        wxf = lw if dx else 1.0 - lw
        valid = ((yy >= 0) & (yy < wv) & (xx >= 0) & (xx < wv)).astype(f32)
        yc = jnp.clip(yy, 0, wv - 1)
        xc = jnp.clip(xx, 0, wv - 1)
        flat = yc * wv + xc + startv
        gidx = (base_row + flat) * (NB * N_HEADS) + bi * N_HEADS + headv
        idx_ref[0, :, c * 128:(c + 1) * 128] = gidx
        w_ref[0, :, c * 128:(c + 1) * 128] = wyf * wxf * valid * awn


def _run_prep(q2, ref16, wq_s, bq_s, woff_s, boff_s, waw_s, baw_s, s_mat, cref):
    return pl.pallas_call(
        _prep_body,
        grid=(NB,),
        in_specs=[
            pl.BlockSpec((BQ, 256), lambda i: (0, 0)),
            pl.BlockSpec((BQ, 16), lambda i: (0, 0)),
            pl.BlockSpec((1, 256, 256), lambda i: (i, 0, 0)),
            pl.BlockSpec((1, 1, 256), lambda i: (i, 0, 0)),
            pl.BlockSpec((1, 256, 256), lambda i: (i, 0, 0)),
            pl.BlockSpec((1, 1, 256), lambda i: (i, 0, 0)),
            pl.BlockSpec((1, 256, 128), lambda i: (i, 0, 0)),
            pl.BlockSpec((1, 1, 128), lambda i: (i, 0, 0)),
            pl.BlockSpec((128, 128), lambda i: (0, 0)),
            pl.BlockSpec((1, 16, 512), lambda i: (i, 0, 0)),
        ],
        out_specs=[
            pl.BlockSpec((1, BQ, 512), lambda i: (i, 0, 0)),
            pl.BlockSpec((1, BQ, 512), lambda i: (i, 0, 0)),
        ],
        out_shape=[
            jax.ShapeDtypeStruct((NB, BQ, 512), jnp.int32),
            jax.ShapeDtypeStruct((NB, BQ, 512), jnp.float32),
        ],
    )(q2, ref16, wq_s, bq_s, woff_s, boff_s, waw_s, baw_s, s_mat, cref)


_VM_ROWS = 512
_VM_GRID = (B * LEN_IN) // _VM_ROWS


def _vmat_body(x_ref, w_ref, b_ref, o_ref):
    o_ref[...] = (jnp.dot(x_ref[...], w_ref[...],
                          preferred_element_type=jnp.float32)
                  + b_ref[...]).astype(jnp.bfloat16)


def _run_vmat(x, wcat, bcat):
    return pl.pallas_call(
        _vmat_body,
        grid=(_VM_GRID,),
        in_specs=[
            pl.BlockSpec((_VM_ROWS, 256), lambda i: (i, 0)),
            pl.BlockSpec((256, 1536), lambda i: (0, 0)),
            pl.BlockSpec((1, 1536), lambda i: (0, 0)),
        ],
        out_specs=pl.BlockSpec((_VM_ROWS, 1536), lambda i: (i, 0)),
        out_shape=jax.ShapeDtypeStruct((B * LEN_IN, 1536), jnp.bfloat16),
    )(x, wcat, bcat)


CH = 5
NCHUNK = ITEMS_PER_WORKER // CH


def _gather_body(table_ref, idx_ref, w_ref, out_ref, idx_c, w_c, rows_v,
                 out_c, sem_s, sem_g, sem_o):
    wid = lax.axis_index("s") * 2 + lax.axis_index("c")
    base = wid * ITEMS_PER_WORKER

    def fire_staging(k, slot):
        g0 = base + k * CH
        pltpu.async_copy(idx_ref.at[pl.ds(g0, CH)], idx_c.at[slot],
                         sem_s.at[slot])
        pltpu.async_copy(w_ref.at[pl.ds(g0, CH)], w_c.at[slot],
                         sem_s.at[slot])

    def fire_gathers(slot, ii, par):
        for c in range(4):
            pltpu.async_copy(table_ref.at[idx_c.at[slot, ii, c]],
                             rows_v.at[par, c], sem_g.at[par])

    def wait_gathers(slot, par):
        for c in range(4):
            pltpu.make_async_copy(table_ref.at[idx_c.at[slot, 0, c]],
                                  rows_v.at[par, c], sem_g.at[par]).wait()

    fire_staging(0, 0)

    def chunk(k, carry):
        slot = k & 1
        g0 = base + k * CH
        pltpu.make_async_copy(idx_ref.at[pl.ds(g0, CH)], idx_c.at[slot],
                              sem_s.at[slot]).wait()
        pltpu.make_async_copy(w_ref.at[pl.ds(g0, CH)], w_c.at[slot],
                              sem_s.at[slot]).wait()

        @pl.when(k + 1 < NCHUNK)
        def _():
            fire_staging(k + 1, 1 - slot)

        @pl.when(k >= 2)
        def _():
            pltpu.make_async_copy(out_c.at[slot], out_ref.at[pl.ds(g0, CH)],
                                  sem_o.at[slot]).wait()

        fire_gathers(slot, 0, 0)

        def item(ii, carry2):
            par = ii & 1

            @pl.when(ii + 1 < CH)
            def _():
                fire_gathers(slot, ii + 1, 1 - par)

            wait_gathers(slot, par)
            acc = [jnp.zeros((16,), jnp.float32) for _ in range(16)]
            for c in range(4):
                for gq in range(8):
                    wv = w_c[slot, ii, pl.ds(c * 128 + gq * 16, 16)]
                    for t in range(16):
                        jb = gq * 16 + t
                        h = (jb >> 2) & 7
                        r = rows_v[par, c, jb, :]
                        a, b = plsc.unpack(
                            r, format=plsc.PackFormat.INTERLEAVED)
                        acc[2 * h] = acc[2 * h] + a * wv[t]
                        acc[2 * h + 1] = acc[2 * h + 1] + b * wv[t]
            for h in range(8):
                out_c[slot, ii, h, pl.ds(0, 16)] = acc[2 * h]
                out_c[slot, ii, h, pl.ds(16, 16)] = acc[2 * h + 1]
            return carry2

        lax.fori_loop(0, CH, item, 0)
        pltpu.async_copy(out_c.at[slot], out_ref.at[pl.ds(g0, CH)],
                         sem_o.at[slot])
        return carry

    lax.fori_loop(0, NCHUNK, chunk, 0)
    for s in range(2):
        pltpu.make_async_copy(out_c.at[s], out_ref.at[pl.ds(base, CH)],
                              sem_o.at[s]).wait()


def _run_gather(table, idx3, w2):
    mesh = plsc.VectorSubcoreMesh(core_axis_name="c", subcore_axis_name="s")
    fn = pl.kernel(
        _gather_body,
        out_type=jax.ShapeDtypeStruct((N_ITEMS, 8, 32), jnp.float32),
        mesh=mesh,
        compiler_params=pltpu.CompilerParams(needs_layout_passes=False,
                                             use_tc_tiling_on_sc=False),
        scratch_types=[
            pltpu.VMEM((2, CH, 4, 128), jnp.int32),
            pltpu.VMEM((2, CH, 512), jnp.float32),
            pltpu.VMEM((2, 4, 128, 32), jnp.bfloat16),
            pltpu.VMEM((2, CH, 8, 32), jnp.float32),
            pltpu.SemaphoreType.DMA((2,)),
            pltpu.SemaphoreType.DMA((2,)),
            pltpu.SemaphoreType.DMA((2,)),
        ],
    )
    return fn(table, idx3, w2)


def _final_body(q_ref, g_ref, wout_ref, bout_ref, wpw_ref, bpw_ref, woc_ref,
                boc_ref, o_ref):
    f32 = jnp.float32
    q = q_ref[...]
    pwl = jnp.dot(q, wpw_ref[...], preferred_element_type=f32) + bpw_ref[...]
    pm = jnp.max(pwl, axis=1, keepdims=True)
    pe = jnp.exp(pwl - pm)
    pw = pe / jnp.sum(pe, axis=1, keepdims=True)
    acc = None
    for bi in range(NB):
        ob = jnp.dot(g_ref[bi], wout_ref[bi],
                     preferred_element_type=f32) + bout_ref[bi]
        t = pw[:, bi:bi + 1] * ob
        acc = t if acc is None else acc + t
    o_ref[...] = jnp.dot(acc, woc_ref[...],
                         preferred_element_type=f32) + boc_ref[...]


def _run_final(q2, g6, wout_s, bout_s, wpw_t, bpw_p, woc_t, boc2):
    return pl.pallas_call(
        _final_body,
        out_shape=jax.ShapeDtypeStruct((BQ, 256), jnp.float32),
    )(q2, g6, wout_s, bout_s, wpw_t, bpw_p, woc_t, boc2)


def kernel(query, reference_points, input_flatten, input_spatial_shapes,
           input_level_start_index, params):
    del input_spatial_shapes, input_level_start_index
    f32 = jnp.float32
    q2 = query.reshape(BQ, 256)
    ref16 = reference_points.reshape(BQ, 16)
    x = input_flatten.reshape(B * LEN_IN, 256)

    eye = jnp.asarray(np.eye(256, dtype=np.float32))
    wq_s = jnp.stack([eye] + [params["W_q_" + s].T for s in SUFS[1:]])
    bq_s = jnp.stack([jnp.zeros((256,), f32)] +
                     [params["b_q_" + s] for s in SUFS[1:]]).reshape(NB, 1, 256)
    woff_s = jnp.stack([params["W_off_" + s][PERM_OFF].T for s in SUFS])
    boff_s = jnp.stack([params["b_off_" + s][PERM_OFF]
                        for s in SUFS]).reshape(NB, 1, 256)
    waw_s = jnp.stack([params["W_aw_" + s][PERM_AW].T for s in SUFS])
    baw_s = jnp.stack([params["b_aw_" + s][PERM_AW]
                       for s in SUFS]).reshape(NB, 1, 128)
    idx, w = _run_prep(q2, ref16, wq_s, bq_s, woff_s, boff_s, waw_s, baw_s,
                       jnp.asarray(S_MAT), jnp.asarray(CREF))

    wcat = jnp.concatenate([params["W_val_" + s] for s in SUFS], axis=0).T
    bcat = jnp.concatenate([params["b_val_" + s]
                            for s in SUFS]).reshape(1, 1536)
    v = _run_vmat(x, wcat, bcat)
    table = v.reshape(N_TABLE_ROWS, 32)

    idx3 = idx.reshape(N_ITEMS, 4, 128)
    w2 = w.reshape(N_ITEMS, 512)
    g = _run_gather(table, idx3, w2)
    g6 = g.reshape(NB, BQ, 256)

    wout_s = jnp.stack([params["W_out_" + s].T[OUT_PERM] for s in SUFS])
    bout_s = jnp.stack([params["b_out_" + s] for s in SUFS]).reshape(NB, 1, 256)
    wpw_t = jnp.concatenate([params["W_pw"].T, jnp.zeros((256, 2), f32)],
                            axis=1)
    bpw_p = jnp.concatenate([params["b_pw"],
                             jnp.full((2,), -1e30, f32)]).reshape(1, 8)
    woc_t = params["W_oc"].T
    boc2 = params["b_oc"].reshape(1, 256)
    out = _run_final(q2, g6, wout_s, bout_s, wpw_t, bpw_p, woc_t, boc2)
    return out.reshape(B, LQ, 256)


# speedup vs baseline: 17.5098x; 1.6712x over previous
import functools

import jax
import jax.numpy as jnp
import numpy as np
from jax import lax
from jax.experimental import pallas as pl
from jax.experimental.pallas import tpu as pltpu
from jax.experimental.pallas import tpu_sc as plsc

D_MODEL = 256
N_HEADS = 8
N_LEVELS = 4
N_POINTS = 4
LEVEL_WH = (128, 64, 32, 16)
LEVEL_START = (0, 16384, 20480, 21504)
LEN_IN = 21760
B = 4
LQ = 300
BQ = B * LQ
SUFS = ("g", "head", "lt", "rt", "ul", "ll")
BOXC = {
    "g": (0.5, 0.5, 1.0, 1.0),
    "head": (0.5, 0.115, 0.7, 0.23),
    "lt": (0.25, 0.41, 0.5, 0.36),
    "rt": (0.75, 0.41, 0.5, 0.36),
    "ul": (0.5, 0.655, 0.7, 0.23),
    "ll": (0.5, 0.885, 0.7, 0.23),
}
NB = len(SUFS)
N_ITEMS = NB * BQ
N_WORKERS = 32
ITEMS_PER_WORKER = N_ITEMS // N_WORKERS
N_TABLE_ROWS = B * LEN_IN * NB * N_HEADS


def _build_consts():
    perm_off = np.zeros(256, np.int64)
    for col in range(256):
        xy = col // 128
        r = col % 128
        l, hp = r // 32, r % 32
        h, p = hp // 4, hp % 4
        perm_off[col] = h * 32 + l * 8 + p * 2 + xy
    perm_aw = np.zeros(128, np.int64)
    for j in range(128):
        l, hp = j // 32, j % 32
        h, p = hp // 4, hp % 4
        perm_aw[j] = h * 16 + l * 4 + p
    s_mat = np.zeros((128, 128), np.float32)
    for j in range(128):
        for jp in range(128):
            if (j % 32) // 4 == (jp % 32) // 4:
                s_mat[jp, j] = 1.0
    cref = np.zeros((NB, 16, 512), np.float32)
    for bi, suf in enumerate(SUFS):
        ax, ay, sw, sh = BOXC[suf]
        m = np.zeros((4, 4), np.float32)
        m[0, 0] = 1.0
        m[1, 1] = 1.0
        m[2, 0] = ax - 0.5
        m[3, 1] = ay - 0.5
        m[2, 2] = sw
        m[3, 3] = sh
        for k in range(4):
            for j in range(128):
                l = j // 32
                for mm in range(4):
                    cref[bi, l * 4 + mm, k * 128 + j] = m[mm, k]
    return perm_off, perm_aw, s_mat, cref


PERM_OFF, PERM_AW, S_MAT, CREF = _build_consts()

VAL_PERM = np.zeros(1536, np.int64)
for _bh in range(48):
    for _k in range(16):
        VAL_PERM[_bh * 16 + _k] = _bh * 32 + _k
        VAL_PERM[768 + _bh * 16 + _k] = _bh * 32 + 16 + _k


def _prep_body(q_ref, ref_ref, wq_ref, bq_ref, woff_ref, boff_ref, waw_ref,
               baw_ref, s_ref, cref_ref, idx_ref, w_ref):
    bi = pl.program_id(0)
    f32 = jnp.float32
    q = q_ref[...]
    qb = jnp.dot(q, wq_ref[0], preferred_element_type=f32, precision=lax.Precision.HIGHEST) + bq_ref[0]
    offp = jnp.dot(qb, woff_ref[0], preferred_element_type=f32, precision=lax.Precision.HIGHEST) + boff_ref[0]
    awl = jnp.dot(qb, waw_ref[0], preferred_element_type=f32, precision=lax.Precision.HIGHEST) + baw_ref[0]
    mx = jnp.max(awl, axis=1, keepdims=True)
    e = jnp.exp(awl - mx)
    den = jnp.dot(e, s_ref[...], preferred_element_type=f32, precision=lax.Precision.HIGHEST)
    awn = e / den
    refq = jnp.dot(ref_ref[...], cref_ref[0], preferred_element_type=f32, precision=lax.Precision.HIGHEST)
    cx, cy = refq[:, 0:128], refq[:, 128:256]
    rw, rh = refq[:, 256:384], refq[:, 384:512]
    offx, offy = offp[:, 0:128], offp[:, 128:256]

    li = lax.broadcasted_iota(jnp.int32, (BQ, 128), 1)
    lvl = lax.shift_right_logical(li, 5)
    wv = lax.shift_right_logical(jnp.full_like(li, 128), lvl)
    startv = jnp.where(lvl == 0, LEVEL_START[0],
                       jnp.where(lvl == 1, LEVEL_START[1],
                                 jnp.where(lvl == 2, LEVEL_START[2],
                                           LEVEL_START[3])))
    headv = lax.shift_right_logical(li & 31, 2)
    bv = lax.broadcasted_iota(jnp.int32, (BQ, 128), 0) // LQ
    wf = wv.astype(f32)

    locx = cx + offx / N_POINTS * rw * 0.5
    locy = cy + offy / N_POINTS * rh * 0.5
    x = locx * wf - 0.5
    y = locy * wf - 0.5
    x0 = jnp.floor(x)
    y0 = jnp.floor(y)
    lw = x - x0
    lh = y - y0
    x0i = x0.astype(jnp.int32)
    y0i = y0.astype(jnp.int32)
    base_row = bv * LEN_IN

    for c, (dy, dx) in enumerate(((0, 0), (0, 1), (1, 0), (1, 1))):
        yy = y0i + dy
        xx = x0i + dx
        wyf = lh if dy else 1.0 - lh
        wxf = lw if dx else 1.0 - lw
        valid = ((yy >= 0) & (yy < wv) & (xx >= 0) & (xx < wv)).astype(f32)
        yc = jnp.clip(yy, 0, wv - 1)
        xc = jnp.clip(xx, 0, wv - 1)
        flat = yc * wv + xc + startv
        gidx = (base_row + flat) * (NB * N_HEADS) + bi * N_HEADS + headv
        idx_ref[0, :, c * 128:(c + 1) * 128] = gidx
        w_ref[0, :, c * 128:(c + 1) * 128] = wyf * wxf * valid * awn


def _run_prep(q2, ref16, wq_s, bq_s, woff_s, boff_s, waw_s, baw_s, s_mat, cref):
    return pl.pallas_call(
        _prep_body,
        grid=(NB,),
        in_specs=[
            pl.BlockSpec((BQ, 256), lambda i: (0, 0)),
            pl.BlockSpec((BQ, 16), lambda i: (0, 0)),
            pl.BlockSpec((1, 256, 256), lambda i: (i, 0, 0)),
            pl.BlockSpec((1, 1, 256), lambda i: (i, 0, 0)),
            pl.BlockSpec((1, 256, 256), lambda i: (i, 0, 0)),
            pl.BlockSpec((1, 1, 256), lambda i: (i, 0, 0)),
            pl.BlockSpec((1, 256, 128), lambda i: (i, 0, 0)),
            pl.BlockSpec((1, 1, 128), lambda i: (i, 0, 0)),
            pl.BlockSpec((128, 128), lambda i: (0, 0)),
            pl.BlockSpec((1, 16, 512), lambda i: (i, 0, 0)),
        ],
        out_specs=[
            pl.BlockSpec((1, BQ, 512), lambda i: (i, 0, 0)),
            pl.BlockSpec((1, BQ, 512), lambda i: (i, 0, 0)),
        ],
        out_shape=[
            jax.ShapeDtypeStruct((NB, BQ, 512), jnp.int32),
            jax.ShapeDtypeStruct((NB, BQ, 512), jnp.float32),
        ],
    )(q2, ref16, wq_s, bq_s, woff_s, boff_s, waw_s, baw_s, s_mat, cref)


_VM_ROWS = 512
_VM_GRID = (B * LEN_IN) // _VM_ROWS


def _rtne_bf16_bits(y):
    r = lax.bitcast_convert_type(y, jnp.uint32)
    c16 = jnp.uint32(16)
    return lax.shift_right_logical(
        r + jnp.uint32(0x7FFF) + (lax.shift_right_logical(r, c16)
                                  & jnp.uint32(1)), c16)


def _vmat_body(x_ref, w_ref, b_ref, o_ref):
    xb = x_ref[...].astype(jnp.bfloat16)
    y = jnp.dot(xb, w_ref[...], preferred_element_type=jnp.float32) + b_ref[...]
    lo = _rtne_bf16_bits(y[:, 0:768])
    hi = _rtne_bf16_bits(y[:, 768:1536])
    o_ref[...] = lo | (hi << 16)


def _run_vmat(x, wcat, bcat):
    return pl.pallas_call(
        _vmat_body,
        grid=(_VM_GRID,),
        in_specs=[
            pl.BlockSpec((_VM_ROWS, 256), lambda i: (i, 0)),
            pl.BlockSpec((256, 1536), lambda i: (0, 0)),
            pl.BlockSpec((1, 1536), lambda i: (0, 0)),
        ],
        out_specs=pl.BlockSpec((_VM_ROWS, 768), lambda i: (i, 0)),
        out_shape=jax.ShapeDtypeStruct((B * LEN_IN, 768), jnp.uint32),
    )(x, wcat, bcat)


CH = 25
NCHUNK = ITEMS_PER_WORKER // CH


def _gather_body(table_ref, idx_ref, w_ref, out_ref, idx_c, w_c, rows_v,
                 out_c, sem_s, sem_g, sem_o):
    wid = lax.axis_index("s") * 2 + lax.axis_index("c")
    base = wid * ITEMS_PER_WORKER

    def fire_staging(k, slot):
        g0 = base + k * CH
        pltpu.async_copy(idx_ref.at[pl.ds(g0, CH)], idx_c.at[slot],
                         sem_s.at[slot])
        pltpu.async_copy(w_ref.at[pl.ds(g0, CH)], w_c.at[slot],
                         sem_s.at[slot])

    def fire_gathers(slot, ii, par):
        for c in range(4):
            pltpu.async_copy(table_ref.at[idx_c.at[slot, ii, c]],
                             rows_v.at[par, c], sem_g.at[par])

    def wait_gathers(slot, par):
        for c in range(4):
            pltpu.make_async_copy(table_ref.at[idx_c.at[slot, 0, c]],
                                  rows_v.at[par, c], sem_g.at[par]).wait()

    fire_staging(0, 0)

    def chunk(k, carry):
        slot = k & 1
        g0 = base + k * CH
        pltpu.make_async_copy(idx_ref.at[pl.ds(g0, CH)], idx_c.at[slot],
                              sem_s.at[slot]).wait()
        pltpu.make_async_copy(w_ref.at[pl.ds(g0, CH)], w_c.at[slot],
                              sem_s.at[slot]).wait()

        @pl.when(k + 1 < NCHUNK)
        def _():
            fire_staging(k + 1, 1 - slot)

        @pl.when(k >= 2)
        def _():
            pltpu.make_async_copy(out_c.at[slot], out_ref.at[pl.ds(g0, CH)],
                                  sem_o.at[slot]).wait()

        fire_gathers(slot, 0, 0)

        def item(ii, carry2):
            par = ii & 1

            @pl.when(ii + 1 < CH)
            def _():
                fire_gathers(slot, ii + 1, 1 - par)

            wait_gathers(slot, par)
            acc = [jnp.zeros((16,), jnp.float32) for _ in range(16)]
            for c in range(4):
                for gq in range(8):
                    wv = w_c[slot, ii, pl.ds(c * 128 + gq * 16, 16)]
                    for t in range(16):
                        jb = gq * 16 + t
                        h = (jb >> 2) & 7
                        r = plsc.bitcast(rows_v[par, c, jb, :],
                                         jnp.bfloat16)
                        a, b = plsc.unpack(
                            r, format=plsc.PackFormat.INTERLEAVED)
                        acc[2 * h] = acc[2 * h] + a * wv[t]
                        acc[2 * h + 1] = acc[2 * h + 1] + b * wv[t]
            for h in range(8):
                out_c[slot, ii, h, pl.ds(0, 16)] = acc[2 * h]
                out_c[slot, ii, h, pl.ds(16, 16)] = acc[2 * h + 1]
            return carry2

        lax.fori_loop(0, CH, item, 0)
        pltpu.async_copy(out_c.at[slot], out_ref.at[pl.ds(g0, CH)],
                         sem_o.at[slot])
        return carry

    lax.fori_loop(0, NCHUNK, chunk, 0)
    for s in range(2):
        pltpu.make_async_copy(out_c.at[s], out_ref.at[pl.ds(base, CH)],
                              sem_o.at[s]).wait()


def _run_gather(table, idx3, w2):
    mesh = plsc.VectorSubcoreMesh(core_axis_name="c", subcore_axis_name="s")
    fn = pl.kernel(
        _gather_body,
        out_type=jax.ShapeDtypeStruct((N_ITEMS, 8, 32), jnp.float32),
        mesh=mesh,
        compiler_params=pltpu.CompilerParams(needs_layout_passes=False,
                                             use_tc_tiling_on_sc=False),
        scratch_types=[
            pltpu.VMEM((2, CH, 4, 128), jnp.int32),
            pltpu.VMEM((2, CH, 512), jnp.float32),
            pltpu.VMEM((2, 4, 128, 16), jnp.uint32),
            pltpu.VMEM((2, CH, 8, 32), jnp.float32),
            pltpu.SemaphoreType.DMA((2,)),
            pltpu.SemaphoreType.DMA((2,)),
            pltpu.SemaphoreType.DMA((2,)),
        ],
    )
    return fn(table, idx3, w2)


def _final_body(q_ref, g_ref, wout_ref, bout_ref, wpw_ref, bpw_ref, woc_ref,
                boc_ref, o_ref):
    f32 = jnp.float32
    q = q_ref[...]
    pwl = jnp.dot(q, wpw_ref[...], preferred_element_type=f32) + bpw_ref[...]
    pm = jnp.max(pwl, axis=1, keepdims=True)
    pe = jnp.exp(pwl - pm)
    pw = pe / jnp.sum(pe, axis=1, keepdims=True)
    acc = None
    for bi in range(NB):
        ob = jnp.dot(g_ref[bi], wout_ref[bi],
                     preferred_element_type=f32) + bout_ref[bi]
        t = pw[:, bi:bi + 1] * ob
        acc = t if acc is None else acc + t
    o_ref[...] = jnp.dot(acc, woc_ref[...],
                         preferred_element_type=f32) + boc_ref[...]


def _run_final(q2, g6, wout_s, bout_s, wpw_t, bpw_p, woc_t, boc2):
    return pl.pallas_call(
        _final_body,
        out_shape=jax.ShapeDtypeStruct((BQ, 256), jnp.float32),
    )(q2, g6, wout_s, bout_s, wpw_t, bpw_p, woc_t, boc2)


def kernel(query, reference_points, input_flatten, input_spatial_shapes,
           input_level_start_index, params):
    del input_spatial_shapes, input_level_start_index
    f32 = jnp.float32
    q2 = query.reshape(BQ, 256)
    ref16 = reference_points.reshape(BQ, 16)
    x = input_flatten.reshape(B * LEN_IN, 256)

    eye = jnp.asarray(np.eye(256, dtype=np.float32))
    wq_s = jnp.stack([eye] + [params["W_q_" + s].T for s in SUFS[1:]])
    bq_s = jnp.stack([jnp.zeros((256,), f32)] +
                     [params["b_q_" + s] for s in SUFS[1:]]).reshape(NB, 1, 256)
    woff_s = jnp.stack([params["W_off_" + s][PERM_OFF].T for s in SUFS])
    boff_s = jnp.stack([params["b_off_" + s][PERM_OFF]
                        for s in SUFS]).reshape(NB, 1, 256)
    waw_s = jnp.stack([params["W_aw_" + s][PERM_AW].T for s in SUFS])
    baw_s = jnp.stack([params["b_aw_" + s][PERM_AW]
                       for s in SUFS]).reshape(NB, 1, 128)
    idx, w = _run_prep(q2, ref16, wq_s, bq_s, woff_s, boff_s, waw_s, baw_s,
                       jnp.asarray(S_MAT), jnp.asarray(CREF))

    wcat = jnp.concatenate([params["W_val_" + s] for s in SUFS],
                           axis=0).T[:, VAL_PERM].astype(jnp.bfloat16)
    bcat = jnp.concatenate([params["b_val_" + s]
                            for s in SUFS])[VAL_PERM].reshape(1, 1536)
    v = _run_vmat(x, wcat, bcat)
    table = v.reshape(N_TABLE_ROWS, 16)

    idx3 = idx.reshape(N_ITEMS, 4, 128)
    w2 = w.reshape(N_ITEMS, 512)
    g = _run_gather(table, idx3, w2)
    g6 = g.reshape(NB, BQ, 256)

    wout_s = jnp.stack([params["W_out_" + s].T for s in SUFS])
    bout_s = jnp.stack([params["b_out_" + s] for s in SUFS]).reshape(NB, 1, 256)
    wpw_t = jnp.concatenate([params["W_pw"].T, jnp.zeros((256, 2), f32)],
                            axis=1)
    bpw_p = jnp.concatenate([params["b_pw"],
                             jnp.full((2,), -1e30, f32)]).reshape(1, 8)
    woc_t = params["W_oc"].T
    boc2 = params["b_oc"].reshape(1, 256)
    out = _run_final(q2, g6, wout_s, bout_s, wpw_t, bpw_p, woc_t, boc2)
    return out.reshape(B, LQ, 256)


# speedup vs baseline: 18.7339x; 1.0699x over previous
import functools

import jax
import jax.numpy as jnp
import numpy as np
from jax import lax
from jax.experimental import pallas as pl
from jax.experimental.pallas import tpu as pltpu
from jax.experimental.pallas import tpu_sc as plsc

D_MODEL = 256
N_HEADS = 8
N_LEVELS = 4
N_POINTS = 4
LEVEL_WH = (128, 64, 32, 16)
LEVEL_START = (0, 16384, 20480, 21504)
LEN_IN = 21760
B = 4
LQ = 300
BQ = B * LQ
SUFS = ("g", "head", "lt", "rt", "ul", "ll")
BOXC = {
    "g": (0.5, 0.5, 1.0, 1.0),
    "head": (0.5, 0.115, 0.7, 0.23),
    "lt": (0.25, 0.41, 0.5, 0.36),
    "rt": (0.75, 0.41, 0.5, 0.36),
    "ul": (0.5, 0.655, 0.7, 0.23),
    "ll": (0.5, 0.885, 0.7, 0.23),
}
NB = len(SUFS)
N_ITEMS = NB * BQ
N_WORKERS = 32
ITEMS_PER_WORKER = N_ITEMS // N_WORKERS
N_TABLE_ROWS = B * LEN_IN * NB * N_HEADS


def _build_consts():
    perm_off = np.zeros(256, np.int64)
    for col in range(256):
        xy = col // 128
        r = col % 128
        l, hp = r // 32, r % 32
        h, p = hp // 4, hp % 4
        perm_off[col] = h * 32 + l * 8 + p * 2 + xy
    perm_aw = np.zeros(128, np.int64)
    for j in range(128):
        l, hp = j // 32, j % 32
        h, p = hp // 4, hp % 4
        perm_aw[j] = h * 16 + l * 4 + p
    s_mat = np.zeros((128, 128), np.float32)
    for j in range(128):
        for jp in range(128):
            if (j % 32) // 4 == (jp % 32) // 4:
                s_mat[jp, j] = 1.0
    cref = np.zeros((NB, 16, 512), np.float32)
    for bi, suf in enumerate(SUFS):
        ax, ay, sw, sh = BOXC[suf]
        m = np.zeros((4, 4), np.float32)
        m[0, 0] = 1.0
        m[1, 1] = 1.0
        m[2, 0] = ax - 0.5
        m[3, 1] = ay - 0.5
        m[2, 2] = sw
        m[3, 3] = sh
        for k in range(4):
            for j in range(128):
                l = j // 32
                for mm in range(4):
                    cref[bi, l * 4 + mm, k * 128 + j] = m[mm, k]
    return perm_off, perm_aw, s_mat, cref


PERM_OFF, PERM_AW, S_MAT, CREF = _build_consts()

VAL_PERM = np.zeros(1536, np.int64)
for _bh in range(48):
    for _k in range(16):
        VAL_PERM[_bh * 16 + _k] = _bh * 32 + _k
        VAL_PERM[768 + _bh * 16 + _k] = _bh * 32 + 16 + _k


def _prep_body(q_ref, ref_ref, wq_ref, bq_ref, woff_ref, boff_ref, waw_ref,
               baw_ref, s_ref, cref_ref, idx_ref, w_ref):
    bi = pl.program_id(0)
    f32 = jnp.float32
    q = q_ref[...]
    qb = jnp.dot(q, wq_ref[0], preferred_element_type=f32, precision=lax.Precision.HIGHEST) + bq_ref[0]
    offp = jnp.dot(qb, woff_ref[0], preferred_element_type=f32, precision=lax.Precision.HIGHEST) + boff_ref[0]
    awl = jnp.dot(qb, waw_ref[0], preferred_element_type=f32, precision=lax.Precision.HIGHEST) + baw_ref[0]
    mx = jnp.max(awl, axis=1, keepdims=True)
    e = jnp.exp(awl - mx)
    den = jnp.dot(e, s_ref[...], preferred_element_type=f32, precision=lax.Precision.HIGHEST)
    awn = e / den
    refq = jnp.dot(ref_ref[...], cref_ref[0], preferred_element_type=f32, precision=lax.Precision.HIGHEST)
    cx, cy = refq[:, 0:128], refq[:, 128:256]
    rw, rh = refq[:, 256:384], refq[:, 384:512]
    offx, offy = offp[:, 0:128], offp[:, 128:256]

    li = lax.broadcasted_iota(jnp.int32, (BQ, 128), 1)
    lvl = lax.shift_right_logical(li, 5)
    wv = lax.shift_right_logical(jnp.full_like(li, 128), lvl)
    startv = jnp.where(lvl == 0, LEVEL_START[0],
                       jnp.where(lvl == 1, LEVEL_START[1],
                                 jnp.where(lvl == 2, LEVEL_START[2],
                                           LEVEL_START[3])))
    headv = lax.shift_right_logical(li & 31, 2)
    bv = lax.broadcasted_iota(jnp.int32, (BQ, 128), 0) // LQ
    wf = wv.astype(f32)

    locx = cx + offx / N_POINTS * rw * 0.5
    locy = cy + offy / N_POINTS * rh * 0.5
    x = locx * wf - 0.5
    y = locy * wf - 0.5
    x0 = jnp.floor(x)
    y0 = jnp.floor(y)
    lw = x - x0
    lh = y - y0
    x0i = x0.astype(jnp.int32)
    y0i = y0.astype(jnp.int32)
    base_row = bv * LEN_IN

    for c, (dy, dx) in enumerate(((0, 0), (0, 1), (1, 0), (1, 1))):
        yy = y0i + dy
        xx = x0i + dx
        wyf = lh if dy else 1.0 - lh
        wxf = lw if dx else 1.0 - lw
        valid = ((yy >= 0) & (yy < wv) & (xx >= 0) & (xx < wv)).astype(f32)
        yc = jnp.clip(yy, 0, wv - 1)
        xc = jnp.clip(xx, 0, wv - 1)
        flat = yc * wv + xc + startv
        gidx = (base_row + flat) * (NB * N_HEADS) + bi * N_HEADS + headv
        idx_ref[0, :, c * 128:(c + 1) * 128] = gidx
        w_ref[0, :, c * 128:(c + 1) * 128] = wyf * wxf * valid * awn


def _run_prep(q2, ref16, wq_s, bq_s, woff_s, boff_s, waw_s, baw_s, s_mat, cref):
    return pl.pallas_call(
        _prep_body,
        grid=(NB,),
        in_specs=[
            pl.BlockSpec((BQ, 256), lambda i: (0, 0)),
            pl.BlockSpec((BQ, 16), lambda i: (0, 0)),
            pl.BlockSpec((1, 256, 256), lambda i: (i, 0, 0)),
            pl.BlockSpec((1, 1, 256), lambda i: (i, 0, 0)),
            pl.BlockSpec((1, 256, 256), lambda i: (i, 0, 0)),
            pl.BlockSpec((1, 1, 256), lambda i: (i, 0, 0)),
            pl.BlockSpec((1, 256, 128), lambda i: (i, 0, 0)),
            pl.BlockSpec((1, 1, 128), lambda i: (i, 0, 0)),
            pl.BlockSpec((128, 128), lambda i: (0, 0)),
            pl.BlockSpec((1, 16, 512), lambda i: (i, 0, 0)),
        ],
        out_specs=[
            pl.BlockSpec((1, BQ, 512), lambda i: (i, 0, 0)),
            pl.BlockSpec((1, BQ, 512), lambda i: (i, 0, 0)),
        ],
        out_shape=[
            jax.ShapeDtypeStruct((NB, BQ, 512), jnp.int32),
            jax.ShapeDtypeStruct((NB, BQ, 512), jnp.float32),
        ],
    )(q2, ref16, wq_s, bq_s, woff_s, boff_s, waw_s, baw_s, s_mat, cref)


_VM_ROWS = 512
_VM_GRID = (B * LEN_IN) // _VM_ROWS


def _rtne_bf16_bits(y):
    r = lax.bitcast_convert_type(y, jnp.uint32)
    c16 = jnp.uint32(16)
    return lax.shift_right_logical(
        r + jnp.uint32(0x7FFF) + (lax.shift_right_logical(r, c16)
                                  & jnp.uint32(1)), c16)


def _vmat_body(x_ref, w_ref, b_ref, o_ref):
    xb = x_ref[...].astype(jnp.bfloat16)
    y = jnp.dot(xb, w_ref[...], preferred_element_type=jnp.float32) + b_ref[...]
    lo = _rtne_bf16_bits(y[:, 0:768])
    hi = _rtne_bf16_bits(y[:, 768:1536])
    o_ref[...] = lo | (hi << 16)


def _run_vmat(x, wcat, bcat):
    return pl.pallas_call(
        _vmat_body,
        grid=(_VM_GRID,),
        in_specs=[
            pl.BlockSpec((_VM_ROWS, 256), lambda i: (i, 0)),
            pl.BlockSpec((256, 1536), lambda i: (0, 0)),
            pl.BlockSpec((1, 1536), lambda i: (0, 0)),
        ],
        out_specs=pl.BlockSpec((_VM_ROWS, 768), lambda i: (i, 0)),
        out_shape=jax.ShapeDtypeStruct((B * LEN_IN, 768), jnp.uint32),
    )(x, wcat, bcat)


CH = 25
NCHUNK = ITEMS_PER_WORKER // CH


def _gather_body(table_ref, idx_ref, w_ref, out_ref, idx_c, w_c, rows_v,
                 out_c, sem_s, sem_g, sem_o):
    wid = lax.axis_index("s") * 2 + lax.axis_index("c")
    base = wid * ITEMS_PER_WORKER

    def fire_staging(k, slot):
        g0 = base + k * CH
        pltpu.async_copy(idx_ref.at[pl.ds(g0, CH)], idx_c.at[slot],
                         sem_s.at[slot])
        pltpu.async_copy(w_ref.at[pl.ds(g0, CH)], w_c.at[slot],
                         sem_s.at[slot])

    def fire_gathers(slot, ii, par):
        for c in range(4):
            pltpu.async_copy(table_ref.at[idx_c.at[slot, ii, c]],
                             rows_v.at[par, c], sem_g.at[par])

    def wait_gathers(slot, par):
        for c in range(4):
            pltpu.make_async_copy(table_ref.at[idx_c.at[slot, 0, c]],
                                  rows_v.at[par, c], sem_g.at[par]).wait()

    NDEEP = 3

    fire_staging(0, 0)

    def chunk(k, carry):
        slot = k & 1
        g0 = base + k * CH
        pltpu.make_async_copy(idx_ref.at[pl.ds(g0, CH)], idx_c.at[slot],
                              sem_s.at[slot]).wait()
        pltpu.make_async_copy(w_ref.at[pl.ds(g0, CH)], w_c.at[slot],
                              sem_s.at[slot]).wait()

        @pl.when(k + 1 < NCHUNK)
        def _():
            fire_staging(k + 1, 1 - slot)

        @pl.when(k >= 2)
        def _():
            pltpu.make_async_copy(out_c.at[slot], out_ref.at[pl.ds(g0, CH)],
                                  sem_o.at[slot]).wait()

        fire_gathers(slot, 0, 0)
        fire_gathers(slot, 1, 1)

        def item(ii, carry2):
            par = lax.rem(ii, NDEEP)

            @pl.when(ii + 2 < CH)
            def _():
                fire_gathers(slot, ii + 2, lax.rem(ii + 2, NDEEP))

            wait_gathers(slot, par)
            acc = [jnp.zeros((16,), jnp.float32) for _ in range(16)]
            for c in range(4):
                for gq in range(8):
                    wv = w_c[slot, ii, pl.ds(c * 128 + gq * 16, 16)]
                    for t in range(16):
                        jb = gq * 16 + t
                        h = (jb >> 2) & 7
                        r = plsc.bitcast(rows_v[par, c, jb, :],
                                         jnp.bfloat16)
                        a, b = plsc.unpack(
                            r, format=plsc.PackFormat.INTERLEAVED)
                        acc[2 * h] = acc[2 * h] + a * wv[t]
                        acc[2 * h + 1] = acc[2 * h + 1] + b * wv[t]
            for h in range(8):
                out_c[slot, ii, h, pl.ds(0, 16)] = acc[2 * h]
                out_c[slot, ii, h, pl.ds(16, 16)] = acc[2 * h + 1]
            return carry2

        lax.fori_loop(0, CH, item, 0)
        pltpu.async_copy(out_c.at[slot], out_ref.at[pl.ds(g0, CH)],
                         sem_o.at[slot])
        return carry

    lax.fori_loop(0, NCHUNK, chunk, 0)
    for s in range(2):
        pltpu.make_async_copy(out_c.at[s], out_ref.at[pl.ds(base, CH)],
                              sem_o.at[s]).wait()


def _run_gather(table, idx3, w2):
    mesh = plsc.VectorSubcoreMesh(core_axis_name="c", subcore_axis_name="s")
    fn = pl.kernel(
        _gather_body,
        out_type=jax.ShapeDtypeStruct((N_ITEMS, 8, 32), jnp.float32),
        mesh=mesh,
        compiler_params=pltpu.CompilerParams(needs_layout_passes=False,
                                             use_tc_tiling_on_sc=False),
        scratch_types=[
            pltpu.VMEM((2, CH, 4, 128), jnp.int32),
            pltpu.VMEM((2, CH, 512), jnp.float32),
            pltpu.VMEM((3, 4, 128, 16), jnp.uint32),
            pltpu.VMEM((2, CH, 8, 32), jnp.float32),
            pltpu.SemaphoreType.DMA((2,)),
            pltpu.SemaphoreType.DMA((3,)),
            pltpu.SemaphoreType.DMA((2,)),
        ],
    )
    return fn(table, idx3, w2)


def _final_body(q_ref, g_ref, wout_ref, bout_ref, wpw_ref, bpw_ref, woc_ref,
                boc_ref, o_ref):
    f32 = jnp.float32
    q = q_ref[...]
    pwl = jnp.dot(q, wpw_ref[...], preferred_element_type=f32) + bpw_ref[...]
    pm = jnp.max(pwl, axis=1, keepdims=True)
    pe = jnp.exp(pwl - pm)
    pw = pe / jnp.sum(pe, axis=1, keepdims=True)
    acc = None
    for bi in range(NB):
        ob = jnp.dot(g_ref[bi], wout_ref[bi],
                     preferred_element_type=f32) + bout_ref[bi]
        t = pw[:, bi:bi + 1] * ob
        acc = t if acc is None else acc + t
    o_ref[...] = jnp.dot(acc, woc_ref[...],
                         preferred_element_type=f32) + boc_ref[...]


def _run_final(q2, g6, wout_s, bout_s, wpw_t, bpw_p, woc_t, boc2):
    return pl.pallas_call(
        _final_body,
        out_shape=jax.ShapeDtypeStruct((BQ, 256), jnp.float32),
    )(q2, g6, wout_s, bout_s, wpw_t, bpw_p, woc_t, boc2)


def kernel(query, reference_points, input_flatten, input_spatial_shapes,
           input_level_start_index, params):
    del input_spatial_shapes, input_level_start_index
    f32 = jnp.float32
    q2 = query.reshape(BQ, 256)
    ref16 = reference_points.reshape(BQ, 16)
    x = input_flatten.reshape(B * LEN_IN, 256)

    eye = jnp.asarray(np.eye(256, dtype=np.float32))
    wq_s = jnp.stack([eye] + [params["W_q_" + s].T for s in SUFS[1:]])
    bq_s = jnp.stack([jnp.zeros((256,), f32)] +
                     [params["b_q_" + s] for s in SUFS[1:]]).reshape(NB, 1, 256)
    woff_s = jnp.stack([params["W_off_" + s][PERM_OFF].T for s in SUFS])
    boff_s = jnp.stack([params["b_off_" + s][PERM_OFF]
                        for s in SUFS]).reshape(NB, 1, 256)
    waw_s = jnp.stack([params["W_aw_" + s][PERM_AW].T for s in SUFS])
    baw_s = jnp.stack([params["b_aw_" + s][PERM_AW]
                       for s in SUFS]).reshape(NB, 1, 128)
    idx, w = _run_prep(q2, ref16, wq_s, bq_s, woff_s, boff_s, waw_s, baw_s,
                       jnp.asarray(S_MAT), jnp.asarray(CREF))

    wcat = jnp.concatenate([params["W_val_" + s] for s in SUFS],
                           axis=0).T[:, VAL_PERM].astype(jnp.bfloat16)
    bcat = jnp.concatenate([params["b_val_" + s]
                            for s in SUFS])[VAL_PERM].reshape(1, 1536)
    v = _run_vmat(x, wcat, bcat)
    table = v.reshape(N_TABLE_ROWS, 16)

    idx3 = idx.reshape(N_ITEMS, 4, 128)
    w2 = w.reshape(N_ITEMS, 512)
    g = _run_gather(table, idx3, w2)
    g6 = g.reshape(NB, BQ, 256)

    wout_s = jnp.stack([params["W_out_" + s].T for s in SUFS])
    bout_s = jnp.stack([params["b_out_" + s] for s in SUFS]).reshape(NB, 1, 256)
    wpw_t = jnp.concatenate([params["W_pw"].T, jnp.zeros((256, 2), f32)],
                            axis=1)
    bpw_p = jnp.concatenate([params["b_pw"],
                             jnp.full((2,), -1e30, f32)]).reshape(1, 8)
    woc_t = params["W_oc"].T
    boc2 = params["b_oc"].reshape(1, 256)
    out = _run_final(q2, g6, wout_s, bout_s, wpw_t, bpw_p, woc_t, boc2)
    return out.reshape(B, LQ, 256)


# speedup vs baseline: 18.7924x; 1.0031x over previous
import jax
import jax.numpy as jnp
import numpy as np
from jax import lax
from jax.experimental import pallas as pl
from jax.experimental.pallas import tpu as pltpu
from jax.experimental.pallas import tpu_sc as plsc

D_MODEL = 256
N_HEADS = 8
N_LEVELS = 4
N_POINTS = 4
LEVEL_WH = (128, 64, 32, 16)
LEVEL_START = (0, 16384, 20480, 21504)
LEN_IN = 21760
B = 4
LQ = 300
BQ = B * LQ
SUFS = ("g", "head", "lt", "rt", "ul", "ll")
BOXC = {
    "g": (0.5, 0.5, 1.0, 1.0),
    "head": (0.5, 0.115, 0.7, 0.23),
    "lt": (0.25, 0.41, 0.5, 0.36),
    "rt": (0.75, 0.41, 0.5, 0.36),
    "ul": (0.5, 0.655, 0.7, 0.23),
    "ll": (0.5, 0.885, 0.7, 0.23),
}
NB = len(SUFS)
N_ITEMS = NB * BQ
N_WORKERS = 32
ITEMS_PER_WORKER = N_ITEMS // N_WORKERS
N_TABLE_ROWS = B * LEN_IN * NB * N_HEADS


def _build_consts():
    perm_off = np.zeros(256, np.int64)
    for col in range(256):
        xy = col // 128
        r = col % 128
        l, hp = r // 32, r % 32
        h, p = hp // 4, hp % 4
        perm_off[col] = h * 32 + l * 8 + p * 2 + xy
    perm_aw = np.zeros(128, np.int64)
    for j in range(128):
        l, hp = j // 32, j % 32
        h, p = hp // 4, hp % 4
        perm_aw[j] = h * 16 + l * 4 + p
    s_mat = np.zeros((128, 128), np.float32)
    for j in range(128):
        for jp in range(128):
            if (j % 32) // 4 == (jp % 32) // 4:
                s_mat[jp, j] = 1.0
    cref = np.zeros((NB, 16, 512), np.float32)
    for bi, suf in enumerate(SUFS):
        ax, ay, sw, sh = BOXC[suf]
        m = np.zeros((4, 4), np.float32)
        m[0, 0] = 1.0
        m[1, 1] = 1.0
        m[2, 0] = ax - 0.5
        m[3, 1] = ay - 0.5
        m[2, 2] = sw
        m[3, 3] = sh
        for k in range(4):
            for j in range(128):
                l = j // 32
                for mm in range(4):
                    cref[bi, l * 4 + mm, k * 128 + j] = m[mm, k]
    return perm_off, perm_aw, s_mat, cref


PERM_OFF, PERM_AW, S_MAT, CREF = _build_consts()

VAL_PERM = np.zeros(1536, np.int64)
for _bh in range(48):
    for _k in range(16):
        VAL_PERM[_bh * 16 + _k] = _bh * 32 + _k
        VAL_PERM[768 + _bh * 16 + _k] = _bh * 32 + 16 + _k


def _prep_body(q_ref, ref_ref, wq_ref, bq_ref, woff_ref, boff_ref, waw_ref,
               baw_ref, s_ref, cref_ref, idx_ref, w_ref):
    bi = pl.program_id(0)
    f32 = jnp.float32
    q = q_ref[...]
    qb = jnp.dot(q, wq_ref[0], preferred_element_type=f32, precision=lax.Precision.HIGHEST) + bq_ref[0]
    offp = jnp.dot(qb, woff_ref[0], preferred_element_type=f32, precision=lax.Precision.HIGHEST) + boff_ref[0]
    awl = jnp.dot(qb, waw_ref[0], preferred_element_type=f32, precision=lax.Precision.HIGHEST) + baw_ref[0]
    mx = jnp.max(awl, axis=1, keepdims=True)
    e = jnp.exp(awl - mx)
    den = jnp.dot(e, s_ref[...], preferred_element_type=f32, precision=lax.Precision.HIGHEST)
    awn = e / den
    refq = jnp.dot(ref_ref[...], cref_ref[0], preferred_element_type=f32, precision=lax.Precision.HIGHEST)
    cx, cy = refq[:, 0:128], refq[:, 128:256]
    rw, rh = refq[:, 256:384], refq[:, 384:512]
    offx, offy = offp[:, 0:128], offp[:, 128:256]

    li = lax.broadcasted_iota(jnp.int32, (BQ, 128), 1)
    lvl = lax.shift_right_logical(li, 5)
    wv = lax.shift_right_logical(jnp.full_like(li, 128), lvl)
    startv = jnp.where(lvl == 0, LEVEL_START[0],
                       jnp.where(lvl == 1, LEVEL_START[1],
                                 jnp.where(lvl == 2, LEVEL_START[2],
                                           LEVEL_START[3])))
    headv = lax.shift_right_logical(li & 31, 2)
    bv = lax.broadcasted_iota(jnp.int32, (BQ, 128), 0) // LQ
    wf = wv.astype(f32)

    locx = cx + offx / N_POINTS * rw * 0.5
    locy = cy + offy / N_POINTS * rh * 0.5
    x = locx * wf - 0.5
    y = locy * wf - 0.5
    x0 = jnp.floor(x)
    y0 = jnp.floor(y)
    lw = x - x0
    lh = y - y0
    x0i = x0.astype(jnp.int32)
    y0i = y0.astype(jnp.int32)
    base_row = bv * LEN_IN

    for c, (dy, dx) in enumerate(((0, 0), (0, 1), (1, 0), (1, 1))):
        yy = y0i + dy
        xx = x0i + dx
        wyf = lh if dy else 1.0 - lh
        wxf = lw if dx else 1.0 - lw
        valid = ((yy >= 0) & (yy < wv) & (xx >= 0) & (xx < wv)).astype(f32)
        yc = jnp.clip(yy, 0, wv - 1)
        xc = jnp.clip(xx, 0, wv - 1)
        flat = yc * wv + xc + startv
        gidx = (base_row + flat) * (NB * N_HEADS) + bi * N_HEADS + headv
        idx_ref[0, :, c * 128:(c + 1) * 128] = gidx
        w_ref[0, :, c * 128:(c + 1) * 128] = wyf * wxf * valid * awn


def _run_prep(q2, ref16, wq_s, bq_s, woff_s, boff_s, waw_s, baw_s, s_mat, cref):
    return pl.pallas_call(
        _prep_body,
        grid=(NB,),
        in_specs=[
            pl.BlockSpec((BQ, 256), lambda i: (0, 0)),
            pl.BlockSpec((BQ, 16), lambda i: (0, 0)),
            pl.BlockSpec((1, 256, 256), lambda i: (i, 0, 0)),
            pl.BlockSpec((1, 1, 256), lambda i: (i, 0, 0)),
            pl.BlockSpec((1, 256, 256), lambda i: (i, 0, 0)),
            pl.BlockSpec((1, 1, 256), lambda i: (i, 0, 0)),
            pl.BlockSpec((1, 256, 128), lambda i: (i, 0, 0)),
            pl.BlockSpec((1, 1, 128), lambda i: (i, 0, 0)),
            pl.BlockSpec((128, 128), lambda i: (0, 0)),
            pl.BlockSpec((1, 16, 512), lambda i: (i, 0, 0)),
        ],
        out_specs=[
            pl.BlockSpec((1, BQ, 512), lambda i: (i, 0, 0)),
            pl.BlockSpec((1, BQ, 512), lambda i: (i, 0, 0)),
        ],
        out_shape=[
            jax.ShapeDtypeStruct((NB, BQ, 512), jnp.int32),
            jax.ShapeDtypeStruct((NB, BQ, 512), jnp.float32),
        ],
    )(q2, ref16, wq_s, bq_s, woff_s, boff_s, waw_s, baw_s, s_mat, cref)


_VM_ROWS = 512
_VM_GRID = (B * LEN_IN) // _VM_ROWS


def _rtne_bf16_bits(y):
    r = lax.bitcast_convert_type(y, jnp.uint32)
    c16 = jnp.uint32(16)
    return lax.shift_right_logical(
        r + jnp.uint32(0x7FFF) + (lax.shift_right_logical(r, c16)
                                  & jnp.uint32(1)), c16)


def _vmat_body(x_ref, w_ref, b_ref, o_ref):
    xb = x_ref[...].astype(jnp.bfloat16)
    y = jnp.dot(xb, w_ref[...], preferred_element_type=jnp.float32) + b_ref[...]
    lo = _rtne_bf16_bits(y[:, 0:768])
    hi = _rtne_bf16_bits(y[:, 768:1536])
    o_ref[...] = lo | (hi << 16)


def _run_vmat(x, wcat, bcat):
    return pl.pallas_call(
        _vmat_body,
        grid=(_VM_GRID,),
        in_specs=[
            pl.BlockSpec((_VM_ROWS, 256), lambda i: (i, 0)),
            pl.BlockSpec((256, 1536), lambda i: (0, 0)),
            pl.BlockSpec((1, 1536), lambda i: (0, 0)),
        ],
        out_specs=pl.BlockSpec((_VM_ROWS, 768), lambda i: (i, 0)),
        out_shape=jax.ShapeDtypeStruct((B * LEN_IN, 768), jnp.uint32),
    )(x, wcat, bcat)


CH = 25
NCHUNK = ITEMS_PER_WORKER // CH


def _gather_body(table_ref, idx_ref, w_ref, out_ref, idx_c, w_c, rows_v,
                 out_c, sem_s, sem_g, sem_o):
    wid = lax.axis_index("s") * 2 + lax.axis_index("c")
    base = wid * ITEMS_PER_WORKER

    def fire_staging(k, slot):
        g0 = base + k * CH
        pltpu.async_copy(idx_ref.at[pl.ds(g0, CH)], idx_c.at[slot],
                         sem_s.at[slot])
        pltpu.async_copy(w_ref.at[pl.ds(g0, CH)], w_c.at[slot],
                         sem_s.at[slot])

    def fire_gathers(slot, ii, par):
        for c in range(4):
            pltpu.async_copy(table_ref.at[idx_c.at[slot, ii, c]],
                             rows_v.at[par, c], sem_g.at[par])

    def wait_gathers(slot, par):
        for c in range(4):
            pltpu.make_async_copy(table_ref.at[idx_c.at[slot, 0, c]],
                                  rows_v.at[par, c], sem_g.at[par]).wait()

    NDEEP = 3

    fire_staging(0, 0)

    def chunk(k, carry):
        slot = k & 1
        g0 = base + k * CH
        pltpu.make_async_copy(idx_ref.at[pl.ds(g0, CH)], idx_c.at[slot],
                              sem_s.at[slot]).wait()
        pltpu.make_async_copy(w_ref.at[pl.ds(g0, CH)], w_c.at[slot],
                              sem_s.at[slot]).wait()

        @pl.when(k + 1 < NCHUNK)
        def _():
            fire_staging(k + 1, 1 - slot)

        @pl.when(k >= 2)
        def _():
            pltpu.make_async_copy(out_c.at[slot], out_ref.at[pl.ds(g0, CH)],
                                  sem_o.at[slot]).wait()

        fire_gathers(slot, 0, 0)
        fire_gathers(slot, 1, 1)

        def item(ii, carry2):
            par = lax.rem(ii, NDEEP)

            @pl.when(ii + 2 < CH)
            def _():
                fire_gathers(slot, ii + 2, lax.rem(ii + 2, NDEEP))

            wait_gathers(slot, par)
            acc = [jnp.zeros((16,), jnp.float32) for _ in range(16)]
            for c in range(4):
                for gq in range(8):
                    wv = w_c[slot, ii, pl.ds(c * 128 + gq * 16, 16)]
                    for t in range(16):
                        jb = gq * 16 + t
                        h = (jb >> 2) & 7
                        r = plsc.bitcast(rows_v[par, c, jb, :],
                                         jnp.bfloat16)
                        a, b = plsc.unpack(
                            r, format=plsc.PackFormat.INTERLEAVED)
                        acc[2 * h] = acc[2 * h] + a * wv[t]
                        acc[2 * h + 1] = acc[2 * h + 1] + b * wv[t]
            for h in range(8):
                out_c[slot, ii, h, pl.ds(0, 16)] = acc[2 * h]
                out_c[slot, ii, h, pl.ds(16, 16)] = acc[2 * h + 1]
            return carry2

        lax.fori_loop(0, CH, item, 0)
        pltpu.async_copy(out_c.at[slot], out_ref.at[pl.ds(g0, CH)],
                         sem_o.at[slot])
        return carry

    lax.fori_loop(0, NCHUNK, chunk, 0)
    for s in range(2):
        pltpu.make_async_copy(out_c.at[s], out_ref.at[pl.ds(base, CH)],
                              sem_o.at[s]).wait()


def _run_gather(table, idx3, w2):
    mesh = plsc.VectorSubcoreMesh(core_axis_name="c", subcore_axis_name="s")
    fn = pl.kernel(
        _gather_body,
        out_type=jax.ShapeDtypeStruct((N_ITEMS, 8, 32), jnp.float32),
        mesh=mesh,
        compiler_params=pltpu.CompilerParams(needs_layout_passes=False,
                                             use_tc_tiling_on_sc=False),
        scratch_types=[
            pltpu.VMEM((2, CH, 4, 128), jnp.int32),
            pltpu.VMEM((2, CH, 512), jnp.float32),
            pltpu.VMEM((3, 4, 128, 16), jnp.uint32),
            pltpu.VMEM((2, CH, 8, 32), jnp.float32),
            pltpu.SemaphoreType.DMA((2,)),
            pltpu.SemaphoreType.DMA((3,)),
            pltpu.SemaphoreType.DMA((2,)),
        ],
    )
    return fn(table, idx3, w2)


def _final_body(q_ref, g_ref, wout_ref, bout_ref, wpw_ref, bpw_ref, woc_ref,
                boc_ref, o_ref):
    f32 = jnp.float32
    q = q_ref[...]
    pwl = jnp.dot(q, wpw_ref[...], preferred_element_type=f32) + bpw_ref[...]
    pm = jnp.max(pwl, axis=1, keepdims=True)
    pe = jnp.exp(pwl - pm)
    pw = pe / jnp.sum(pe, axis=1, keepdims=True)
    acc = None
    for bi in range(NB):
        ob = jnp.dot(g_ref[bi], wout_ref[bi],
                     preferred_element_type=f32) + bout_ref[bi]
        t = pw[:, bi:bi + 1] * ob
        acc = t if acc is None else acc + t
    o_ref[...] = jnp.dot(acc, woc_ref[...],
                         preferred_element_type=f32) + boc_ref[...]


def _run_final(q2, g6, wout_s, bout_s, wpw_t, bpw_p, woc_t, boc2):
    return pl.pallas_call(
        _final_body,
        out_shape=jax.ShapeDtypeStruct((BQ, 256), jnp.float32),
    )(q2, g6, wout_s, bout_s, wpw_t, bpw_p, woc_t, boc2)


def kernel(query, reference_points, input_flatten, input_spatial_shapes,
           input_level_start_index, params):
    del input_spatial_shapes, input_level_start_index
    f32 = jnp.float32
    q2 = query.reshape(BQ, 256)
    ref16 = reference_points.reshape(BQ, 16)
    x = input_flatten.reshape(B * LEN_IN, 256)

    eye = jnp.asarray(np.eye(256, dtype=np.float32))
    wq_s = jnp.stack([eye] + [params["W_q_" + s].T for s in SUFS[1:]])
    bq_s = jnp.stack([jnp.zeros((256,), f32)] +
                     [params["b_q_" + s] for s in SUFS[1:]]).reshape(NB, 1, 256)
    woff_s = jnp.stack([params["W_off_" + s][PERM_OFF].T for s in SUFS])
    boff_s = jnp.stack([params["b_off_" + s][PERM_OFF]
                        for s in SUFS]).reshape(NB, 1, 256)
    waw_s = jnp.stack([params["W_aw_" + s][PERM_AW].T for s in SUFS])
    baw_s = jnp.stack([params["b_aw_" + s][PERM_AW]
                       for s in SUFS]).reshape(NB, 1, 128)
    idx, w = _run_prep(q2, ref16, wq_s, bq_s, woff_s, boff_s, waw_s, baw_s,
                       jnp.asarray(S_MAT), jnp.asarray(CREF))

    wcat = jnp.concatenate([params["W_val_" + s] for s in SUFS],
                           axis=0).T[:, VAL_PERM].astype(jnp.bfloat16)
    bcat = jnp.concatenate([params["b_val_" + s]
                            for s in SUFS])[VAL_PERM].reshape(1, 1536)
    v = _run_vmat(x, wcat, bcat)
    table = v.reshape(N_TABLE_ROWS, 16)

    idx3 = idx.reshape(N_ITEMS, 4, 128)
    w2 = w.reshape(N_ITEMS, 512)
    g = _run_gather(table, idx3, w2)
    g6 = g.reshape(NB, BQ, 256)

    wout_s = jnp.stack([params["W_out_" + s].T for s in SUFS])
    bout_s = jnp.stack([params["b_out_" + s] for s in SUFS]).reshape(NB, 1, 256)
    wpw_t = jnp.concatenate([params["W_pw"].T, jnp.zeros((256, 2), f32)],
                            axis=1)
    bpw_p = jnp.concatenate([params["b_pw"],
                             jnp.full((2,), -1e30, f32)]).reshape(1, 8)
    woc_t = params["W_oc"].T
    boc2 = params["b_oc"].reshape(1, 256)
    out = _run_final(q2, g6, wout_s, bout_s, wpw_t, bpw_p, woc_t, boc2)
    return out.reshape(B, LQ, 256)
